# baseline (speedup 1.0000x reference)
import jax
import jax.numpy as jnp
from jax.experimental import pallas as pl
from jax.experimental.pallas import tpu as pltpu

N = 4096
NFEAT = 4096
NCLASS = 8
BLK1 = 1024
BLK2 = 1024


def _support_kernel(x_ref, wgc_ref, sup_ref):
    sup_ref[...] = jnp.dot(x_ref[...], wgc_ref[...],
                           preferred_element_type=jnp.float32)


def _agg_kernel(adj_ref, sup_ref, bgc_ref, wlin_ref, blin_ref,
                ne_ref, y_ref):
    i = pl.program_id(0)
    nb = pl.num_programs(0)

    gc = jnp.dot(adj_ref[...], sup_ref[...],
                 preferred_element_type=jnp.float32)
    ne = jnp.maximum(gc + bgc_ref[...], 0.0)
    ne_ref[...] = ne
    m = jnp.max(ne, axis=1, keepdims=True)
    ls = ne - m - jnp.log(jnp.sum(jnp.exp(ne - m), axis=1, keepdims=True))
    part = jnp.sum(ls * wlin_ref[...], axis=0, keepdims=True)

    @pl.when(i == 0)
    def _init_y():
        y_ref[...] = part

    @pl.when(i > 0)
    def _acc_y():
        y_ref[...] += part

    @pl.when(i == nb - 1)
    def _final_y():
        y_ref[...] += blin_ref[...]


@jax.jit
def kernel(x, adj, W_gc, b_gc, W_lin, b_lin):
    bgc2 = b_gc.reshape(1, NCLASS)
    wlin_t = W_lin.reshape(NFEAT, 1)
    blin2 = b_lin.reshape(1, 1)

    support = pl.pallas_call(
        _support_kernel,
        grid=(N // BLK1,),
        in_specs=[
            pl.BlockSpec((BLK1, NFEAT), lambda i: (i, 0)),
            pl.BlockSpec((NFEAT, NCLASS), lambda i: (0, 0)),
        ],
        out_specs=pl.BlockSpec((BLK1, NCLASS), lambda i: (i, 0)),
        out_shape=jax.ShapeDtypeStruct((N, NCLASS), jnp.float32),
    )(x, W_gc)

    ne, y = pl.pallas_call(
        _agg_kernel,
        grid=(N // BLK2,),
        in_specs=[
            pl.BlockSpec((BLK2, N), lambda i: (i, 0)),
            pl.BlockSpec((N, NCLASS), lambda i: (0, 0)),
            pl.BlockSpec((1, NCLASS), lambda i: (0, 0)),
            pl.BlockSpec((BLK2, 1), lambda i: (i, 0)),
            pl.BlockSpec((1, 1), lambda i: (0, 0)),
        ],
        out_specs=[
            pl.BlockSpec((BLK2, NCLASS), lambda i: (i, 0)),
            pl.BlockSpec((1, NCLASS), lambda i: (0, 0)),
        ],
        out_shape=[
            jax.ShapeDtypeStruct((N, NCLASS), jnp.float32),
            jax.ShapeDtypeStruct((1, NCLASS), jnp.float32),
        ],
    )(adj, support, bgc2, wlin_t, blin2)
    return (y, ne)
